# baseline (speedup 1.0000x reference)
import functools

import jax
import jax.numpy as jnp
from jax import lax
from jax.experimental import pallas as pl
from jax.experimental.pallas import tpu as pltpu
from jax.experimental.pallas import tpu_sc as plsc

CAP = 2000
D = 4096
B = 512
NOISE_SCALE = 0.15

NC = 2
NS = 16
NW = NC * NS
BPW = B // NW
RCH = 4
NCHUNK = BPW // RCH
NSLOT = 3
CBLK = 8
RBLK = 64


def _sc_body(buf_hbm, idx_hbm, noise_hbm, tmp_hbm, *scratch):
    idx_v = scratch[0]
    rows = scratch[1:1 + NSLOT]
    nois = scratch[1 + NSLOT:1 + 2 * NSLOT]
    gsem = scratch[1 + 2 * NSLOT:1 + 3 * NSLOT]
    nsem = scratch[1 + 3 * NSLOT:1 + 4 * NSLOT]
    osem = scratch[1 + 4 * NSLOT:1 + 5 * NSLOT]

    cid = lax.axis_index("c")
    sid = lax.axis_index("s")
    wid = sid * NC + cid
    base = wid * BPW
    pltpu.sync_copy(idx_hbm.at[wid], idx_v)

    ghandle = [None] * NSLOT
    nhandle = [None] * NSLOT
    ohandle = [None] * NSLOT

    def issue(k):
        s = k % NSLOT
        ghandle[s] = pltpu.async_copy(buf_hbm.at[idx_v.at[k]], rows[s], gsem[s])
        nhandle[s] = pltpu.async_copy(
            noise_hbm.at[pl.ds(base + k * RCH, RCH)], nois[s], nsem[s])

    for k in range(min(NSLOT, NCHUNK)):
        issue(k)

    for k in range(NCHUNK):
        s = k % NSLOT
        ghandle[s].wait()
        nhandle[s].wait()
        for r in range(RCH):
            def cbody(j, _, r=r, s=s):
                c = j * 16
                rows[s][r, pl.ds(c, 16)] = (
                    rows[s][r, pl.ds(c, 16)]
                    + nois[s][r, pl.ds(c, 16)] * NOISE_SCALE)
                return 0
            lax.fori_loop(0, D // 16, cbody, 0, unroll=8)
        ohandle[s] = pltpu.async_copy(
            rows[s], tmp_hbm.at[pl.ds(base + k * RCH, RCH)], osem[s])
        nxt = k + NSLOT
        if nxt < NCHUNK:
            ohandle[s].wait()
            issue(nxt)
    for k in range(max(NCHUNK - NSLOT, 0), NCHUNK):
        ohandle[k % NSLOT].wait()


def _stats_body(buf_ref, mean_ref, rstd_ref):
    x = buf_ref[...]
    n = jnp.float32(CAP)
    m = jnp.sum(x, axis=0) / n
    d = x - m[None, :]
    var = jnp.sum(d * d, axis=0) / (n - 1.0)
    std = jnp.maximum(jnp.sqrt(var), 1e-6)
    mean_ref[...] = m
    rstd_ref[...] = 1.0 / std


def _norm_body(tmp_ref, mean_ref, rstd_ref, out_ref):
    out_ref[...] = (tmp_ref[...] - mean_ref[...][None, :]) * rstd_ref[...][None, :]


def kernel(buffer, indices, noise):
    idx3 = jnp.reshape(indices, (NW, NCHUNK, RCH))

    mesh = plsc.VectorSubcoreMesh(core_axis_name="c", subcore_axis_name="s")
    scratch = [pltpu.VMEM((NCHUNK, RCH), jnp.int32)]
    scratch += [pltpu.VMEM((RCH, D), jnp.float32) for _ in range(2 * NSLOT)]
    scratch += [pltpu.SemaphoreType.DMA for _ in range(3 * NSLOT)]
    sc = functools.partial(
        pl.kernel,
        mesh=mesh,
        out_type=jax.ShapeDtypeStruct((B, D), jnp.float32),
        scratch_types=scratch,
    )(_sc_body)
    tmp = sc(buffer, idx3, noise)

    mean, rstd = pl.pallas_call(
        _stats_body,
        grid=(CBLK,),
        in_specs=[pl.BlockSpec((CAP, D // CBLK), lambda i: (0, i))],
        out_specs=[pl.BlockSpec((D // CBLK,), lambda i: (i,)),
                   pl.BlockSpec((D // CBLK,), lambda i: (i,))],
        out_shape=[jax.ShapeDtypeStruct((D,), jnp.float32),
                   jax.ShapeDtypeStruct((D,), jnp.float32)],
    )(buffer)

    return pl.pallas_call(
        _norm_body,
        grid=(B // RBLK,),
        in_specs=[pl.BlockSpec((RBLK, D), lambda i: (i, 0)),
                  pl.BlockSpec((D,), lambda i: (0,)),
                  pl.BlockSpec((D,), lambda i: (0,))],
        out_specs=pl.BlockSpec((RBLK, D), lambda i: (i, 0)),
        out_shape=jax.ShapeDtypeStruct((B, D), jnp.float32),
    )(tmp, mean, rstd)


# speedup vs baseline: 1.4269x; 1.4269x over previous
import functools

import jax
import jax.numpy as jnp
from jax import lax
from jax.experimental import pallas as pl
from jax.experimental.pallas import tpu as pltpu
from jax.experimental.pallas import tpu_sc as plsc

CAP = 2000
D = 4096
B = 512
NOISE_SCALE = 0.15

NC = 2
NS = 16
NW = NC * NS
BPW = B // NW
RCH = 8
NCHUNK = BPW // RCH
CBLK = 8
RBLK = 64


def _sc_body(buf_hbm, idx_hbm, tmp_hbm, *scratch):
    idx_v = scratch[0]
    rows = scratch[1:1 + NCHUNK]
    gsem = scratch[1 + NCHUNK:1 + 2 * NCHUNK]
    osem = scratch[1 + 2 * NCHUNK:1 + 3 * NCHUNK]

    cid = lax.axis_index("c")
    sid = lax.axis_index("s")
    wid = sid * NC + cid
    base = wid * BPW
    pltpu.sync_copy(idx_hbm.at[wid], idx_v)

    gh = [None] * NCHUNK
    for k in range(NCHUNK):
        gh[k] = pltpu.async_copy(buf_hbm.at[idx_v.at[k]], rows[k], gsem[k])
    oh = [None] * NCHUNK
    for k in range(NCHUNK):
        gh[k].wait()
        oh[k] = pltpu.async_copy(
            rows[k], tmp_hbm.at[pl.ds(base + k * RCH, RCH)], osem[k])
    for k in range(NCHUNK):
        oh[k].wait()


def _stats_body(buf_ref, mean_ref, rstd_ref):
    x = buf_ref[...]
    n = jnp.float32(CAP)
    m = jnp.sum(x, axis=0) / n
    d = x - m[None, :]
    var = jnp.sum(d * d, axis=0) / (n - 1.0)
    std = jnp.maximum(jnp.sqrt(var), 1e-6)
    mean_ref[...] = m
    rstd_ref[...] = 1.0 / std


def _norm_body(tmp_ref, noise_ref, mean_ref, rstd_ref, out_ref):
    out_ref[...] = (
        tmp_ref[...] + noise_ref[...] * NOISE_SCALE - mean_ref[...][None, :]
    ) * rstd_ref[...][None, :]


def kernel(buffer, indices, noise):
    idx3 = jnp.reshape(indices, (NW, NCHUNK, RCH))

    mesh = plsc.VectorSubcoreMesh(core_axis_name="c", subcore_axis_name="s")
    scratch = [pltpu.VMEM((NCHUNK, RCH), jnp.int32)]
    scratch += [pltpu.VMEM((RCH, D), jnp.float32) for _ in range(NCHUNK)]
    scratch += [pltpu.SemaphoreType.DMA for _ in range(2 * NCHUNK)]
    sc = functools.partial(
        pl.kernel,
        mesh=mesh,
        out_type=jax.ShapeDtypeStruct((B, D), jnp.float32),
        scratch_types=scratch,
    )(_sc_body)
    tmp = sc(buffer, idx3)

    mean, rstd = pl.pallas_call(
        _stats_body,
        grid=(CBLK,),
        in_specs=[pl.BlockSpec((CAP, D // CBLK), lambda i: (0, i))],
        out_specs=[pl.BlockSpec((D // CBLK,), lambda i: (i,)),
                   pl.BlockSpec((D // CBLK,), lambda i: (i,))],
        out_shape=[jax.ShapeDtypeStruct((D,), jnp.float32),
                   jax.ShapeDtypeStruct((D,), jnp.float32)],
    )(buffer)

    return pl.pallas_call(
        _norm_body,
        grid=(B // RBLK,),
        in_specs=[pl.BlockSpec((RBLK, D), lambda i: (i, 0)),
                  pl.BlockSpec((RBLK, D), lambda i: (i, 0)),
                  pl.BlockSpec((D,), lambda i: (0,)),
                  pl.BlockSpec((D,), lambda i: (0,))],
        out_specs=pl.BlockSpec((RBLK, D), lambda i: (i, 0)),
        out_shape=jax.ShapeDtypeStruct((B, D), jnp.float32),
    )(tmp, noise, mean, rstd)
